# Initial kernel scaffold; baseline (speedup 1.0000x reference)
import jax
import jax.numpy as jnp
from jax.experimental import pallas as pl


def kernel(node_emb, batch_full, dataset_ids, W_forces, w_energy):
    raise NotImplementedError("write your pallas kernel here")


# speedup vs baseline: 10.2068x; 10.2068x over previous
import functools

import jax
import jax.numpy as jnp
from jax import lax
from jax.experimental import pallas as pl
from jax.experimental.pallas import tpu as pltpu
from jax.experimental.pallas import tpu_sc as plsc

N_ATOMS = 524288
N_SYSTEMS = 8192
D_FEAT = 128

NW = 32
CHUNK = N_ATOMS // NW
SUB = 8192
NSUB = CHUNK // SUB

MM_BLOCK = 8192


def _mm_body(emb_ref, w_ref, out_ref):
    out_ref[...] = jnp.dot(emb_ref[...], w_ref[...],
                           preferred_element_type=jnp.float32)


_mm_call = pl.pallas_call(
    _mm_body,
    grid=(N_ATOMS // MM_BLOCK,),
    in_specs=[
        pl.BlockSpec((MM_BLOCK, D_FEAT), lambda i: (i, 0)),
        pl.BlockSpec((D_FEAT, 4), lambda i: (0, 0)),
    ],
    out_specs=pl.BlockSpec((MM_BLOCK, 4), lambda i: (i, 0)),
    out_shape=jax.ShapeDtypeStruct((N_ATOMS, 4), jnp.float32),
    compiler_params=pltpu.CompilerParams(
        dimension_semantics=("arbitrary",)),
)


def _comb_body(part_ref, ds_ref, e0_ref, e1_ref):
    energy = jnp.sum(part_ref[...], axis=0)
    ds = ds_ref[...]
    zero = jnp.zeros_like(energy)
    e0_ref[...] = jnp.where(ds == 0, energy, zero)
    e1_ref[...] = jnp.where(ds == 1, energy, zero)


_comb_call = pl.pallas_call(
    _comb_body,
    out_shape=[jax.ShapeDtypeStruct((N_SYSTEMS,), jnp.float32)] * 2,
)

_sc_mesh = plsc.VectorSubcoreMesh(core_axis_name="c", subcore_axis_name="s")


@functools.partial(
    pl.kernel,
    mesh=_sc_mesh,
    compiler_params=pltpu.CompilerParams(needs_layout_passes=False),
    out_type=[
        jax.ShapeDtypeStruct((3 * N_ATOMS,), jnp.float32),
        jax.ShapeDtypeStruct((3 * N_ATOMS,), jnp.float32),
        jax.ShapeDtypeStruct((NW, N_SYSTEMS), jnp.float32),
    ],
    scratch_types=[
        pltpu.VMEM((N_SYSTEMS,), jnp.int32),
        pltpu.VMEM((SUB + 16,), jnp.int32),
        pltpu.VMEM((SUB * 4,), jnp.float32),
        pltpu.VMEM((SUB,), jnp.float32),
        pltpu.VMEM((SUB * 3,), jnp.float32),
        pltpu.VMEM((SUB * 3,), jnp.float32),
        pltpu.VMEM((N_SYSTEMS,), jnp.float32),
        pltpu.VMEM((16,), jnp.float32),
    ],
)
def _sc_route(in4_hbm, b_hbm, ds_hbm, f0_hbm, f1_hbm, part_hbm,
              ds_v, b_v, in4_v, m_v, f0_v, f1_v, acc, s16):
    wid = lax.axis_index("s") * 2 + lax.axis_index("c")
    iota = lax.iota(jnp.int32, 16)
    zero16 = jnp.zeros((16,), jnp.float32)
    one16 = jnp.ones((16,), jnp.float32)
    izero16 = jnp.zeros((16,), jnp.int32)
    ione16 = jnp.ones((16,), jnp.int32)
    i15_16 = jnp.full((16,), 15, jnp.int32)
    ineg16 = jnp.full((16,), -1, jnp.int32)

    def _ifull(x):
        return jnp.full((16,), x, jnp.int32)

    def zbody(i, carry):
        acc[pl.ds(i * 16, 16)] = zero16
        return carry
    lax.fori_loop(0, N_SYSTEMS // 16, zbody, 0)

    pltpu.sync_copy(ds_hbm, ds_v)

    a_pat = [(16 * k + iota) // 3 for k in range(3)]
    p_pat = [4 * ((16 * k + iota) // 3) + ((16 * k + iota) % 3)
             for k in range(3)]
    e_src = 4 * iota + 3

    for sc in range(NSUB):
        base = wid * CHUNK + sc * SUB
        pltpu.sync_copy(b_hbm.at[pl.ds(base, SUB)], b_v.at[pl.ds(0, SUB)])
        b_v[pl.ds(SUB, 16)] = ineg16
        pltpu.sync_copy(in4_hbm.at[pl.ds(base * 4, SUB * 4)], in4_v)

        def megroup(i, carry):
            b = b_v[pl.ds(i * 16, 16)]
            dsid = plsc.load_gather(ds_v, [b])
            m = jnp.where(dsid == izero16, one16, zero16)
            m_v[pl.ds(i * 16, 16)] = m
            v = plsc.load_gather(in4_v, [e_src + _ifull(i * 64)])
            b_next = plsc.load_gather(b_v, [iota + _ifull(i * 16 + 1)])
            b_prev = plsc.load_gather(
                b_v, [jnp.maximum(iota + _ifull(i * 16 - 1), izero16)])
            g = plsc.cummax(jnp.where(b != b_prev, iota, izero16))
            svec = plsc.cumsum(v)
            s16[...] = svec
            sprev = plsc.load_gather(
                s16, [jnp.maximum(g - ione16, izero16)])
            run = svec - jnp.where(g > izero16, sprev, zero16)
            b_nx = jnp.where(iota == i15_16, ineg16, b_next)
            plsc.addupdate_scatter(acc, [b], run, mask=b != b_nx)
            return carry
        lax.fori_loop(0, SUB // 16, megroup, 0)

        def fgroup(i, carry):
            for k in range(3):
                ma = plsc.load_gather(m_v, [a_pat[k] + _ifull(i * 16)])
                v = plsc.load_gather(in4_v, [p_pat[k] + _ifull(i * 64)])
                f0 = v * ma
                f1 = v - f0
                off = i * 48 + k * 16
                f0_v[pl.ds(off, 16)] = f0
                f1_v[pl.ds(off, 16)] = f1
            return carry
        lax.fori_loop(0, SUB // 16, fgroup, 0)

        pltpu.sync_copy(f0_v, f0_hbm.at[pl.ds(base * 3, SUB * 3)])
        pltpu.sync_copy(f1_v, f1_hbm.at[pl.ds(base * 3, SUB * 3)])

    pltpu.sync_copy(acc, part_hbm.at[wid])


def kernel(node_emb, batch_full, dataset_ids, W_forces, w_energy):
    w_cat = jnp.concatenate([W_forces, w_energy[:, None]], axis=1)
    batch = batch_full.astype(jnp.int32)
    ds = dataset_ids.astype(jnp.int32)

    out4 = _mm_call(node_emb, w_cat)
    f0f, f1f, part = _sc_route(out4.reshape(-1), batch, ds)
    e0, e1 = _comb_call(part, ds)
    return (e0, f0f.reshape(N_ATOMS, 3), e1, f1f.reshape(N_ATOMS, 3))


# speedup vs baseline: 55.0207x; 5.3906x over previous
import functools

import jax
import jax.numpy as jnp
from jax import lax
from jax.experimental import pallas as pl
from jax.experimental.pallas import tpu as pltpu
from jax.experimental.pallas import tpu_sc as plsc

N_ATOMS = 524288
N_SYSTEMS = 8192
D_FEAT = 128
NGRP = N_ATOMS // 128

NW = 32
CHUNK = N_ATOMS // NW
SUB = 4096
NSUB = CHUNK // SUB

MM_BLOCK = 8192
MM_G = MM_BLOCK // 128


def _mm_body(emb_ref, w_ref, out_ref):
    mm_t = lax.dot_general(
        w_ref[...], emb_ref[...],
        dimension_numbers=(((0,), (1,)), ((), ())),
        preferred_element_type=jnp.float32,
    )
    out_ref[...] = jnp.swapaxes(
        mm_t.reshape(8, MM_G, 128), 0, 1)


_mm_call = pl.pallas_call(
    _mm_body,
    grid=(N_ATOMS // MM_BLOCK,),
    in_specs=[
        pl.BlockSpec((MM_BLOCK, D_FEAT), lambda i: (i, 0)),
        pl.BlockSpec((D_FEAT, 8), lambda i: (0, 0)),
    ],
    out_specs=pl.BlockSpec((MM_G, 8, 128), lambda i: (i, 0, 0)),
    out_shape=jax.ShapeDtypeStruct((NGRP, 8, 128), jnp.float32),
    compiler_params=pltpu.CompilerParams(
        dimension_semantics=("arbitrary",)),
)


def _comb_body(part_ref, ds_ref, e0_ref, e1_ref):
    energy = jnp.sum(part_ref[...], axis=0)
    ds = ds_ref[...]
    zero = jnp.zeros_like(energy)
    e0_ref[...] = jnp.where(ds == 0, energy, zero)
    e1_ref[...] = jnp.where(ds == 1, energy, zero)


_comb_call = pl.pallas_call(
    _comb_body,
    out_shape=[jax.ShapeDtypeStruct((N_SYSTEMS,), jnp.float32)] * 2,
)

_sc_mesh = plsc.VectorSubcoreMesh(core_axis_name="c", subcore_axis_name="s")


@functools.partial(
    pl.kernel,
    mesh=_sc_mesh,
    compiler_params=pltpu.CompilerParams(needs_layout_passes=False),
    out_type=[
        jax.ShapeDtypeStruct((NGRP * 512,), jnp.float32),
        jax.ShapeDtypeStruct((NGRP * 512,), jnp.float32),
        jax.ShapeDtypeStruct((NW, N_SYSTEMS), jnp.float32),
    ],
    scratch_types=[
        pltpu.VMEM((N_SYSTEMS,), jnp.int32),
        pltpu.VMEM((SUB + 16,), jnp.int32),
        pltpu.VMEM((SUB * 8,), jnp.float32),
        pltpu.VMEM((SUB * 4,), jnp.float32),
        pltpu.VMEM((SUB * 4,), jnp.float32),
        pltpu.VMEM((N_SYSTEMS,), jnp.float32),
        pltpu.VMEM((16,), jnp.float32),
    ],
)
def _sc_route(in8_hbm, b_hbm, ds_hbm, f0_hbm, f1_hbm, part_hbm,
              ds_v, b_v, in8_v, f0_v, f1_v, acc, s16):
    wid = lax.axis_index("s") * 2 + lax.axis_index("c")
    iota = lax.iota(jnp.int32, 16)
    zero16 = jnp.zeros((16,), jnp.float32)
    one16 = jnp.ones((16,), jnp.float32)
    izero16 = jnp.zeros((16,), jnp.int32)
    ione16 = jnp.ones((16,), jnp.int32)
    i15_16 = jnp.full((16,), 15, jnp.int32)
    ineg16 = jnp.full((16,), -1, jnp.int32)

    def _ifull(x):
        return jnp.full((16,), x, jnp.int32)

    def zbody(i, carry):
        acc[pl.ds(i * 16, 16)] = zero16
        return carry
    lax.fori_loop(0, N_SYSTEMS // 16, zbody, 0)

    pltpu.sync_copy(ds_hbm, ds_v)

    for sc in range(NSUB):
        base = wid * CHUNK + sc * SUB
        pltpu.sync_copy(b_hbm.at[pl.ds(base, SUB)], b_v.at[pl.ds(0, SUB)])
        b_v[pl.ds(SUB, 16)] = ineg16
        pltpu.sync_copy(in8_hbm.at[pl.ds(base * 8, SUB * 8)], in8_v)

        def group(i, carry):
            src = (i // 8) * 1024 + (i % 8) * 16
            dst = (i // 8) * 512 + (i % 8) * 16
            b = b_v[pl.ds(i * 16, 16)]
            dsid = plsc.load_gather(ds_v, [b])
            m = jnp.where(dsid == izero16, one16, zero16)
            for c in range(3):
                v = in8_v[pl.ds(src + c * 128, 16)]
                f0 = v * m
                f0_v[pl.ds(dst + c * 128, 16)] = f0
                f1_v[pl.ds(dst + c * 128, 16)] = v - f0
            e = in8_v[pl.ds(src + 384, 16)]
            b_next = plsc.load_gather(b_v, [iota + _ifull(i * 16 + 1)])
            b_prev = plsc.load_gather(
                b_v, [jnp.maximum(iota + _ifull(i * 16 - 1), izero16)])
            g = plsc.cummax(jnp.where(b != b_prev, iota, izero16))
            svec = plsc.cumsum(e)
            s16[...] = svec
            sprev = plsc.load_gather(
                s16, [jnp.maximum(g - ione16, izero16)])
            run = svec - jnp.where(g > izero16, sprev, zero16)
            b_nx = jnp.where(iota == i15_16, ineg16, b_next)
            plsc.addupdate_scatter(acc, [b], run, mask=b != b_nx)
            return carry
        lax.fori_loop(0, SUB // 16, group, 0)

        pltpu.sync_copy(f0_v, f0_hbm.at[pl.ds(base * 4, SUB * 4)])
        pltpu.sync_copy(f1_v, f1_hbm.at[pl.ds(base * 4, SUB * 4)])

    pltpu.sync_copy(acc, part_hbm.at[wid])


def kernel(node_emb, batch_full, dataset_ids, W_forces, w_energy):
    w8 = jnp.concatenate(
        [W_forces, w_energy[:, None],
         jnp.zeros((D_FEAT, 4), jnp.float32)], axis=1)
    batch = batch_full.astype(jnp.int32)
    ds = dataset_ids.astype(jnp.int32)

    blk = _mm_call(node_emb, w8)
    f0b, f1b, part = _sc_route(blk.reshape(-1), batch, ds)
    e0, e1 = _comb_call(part, ds)

    def _to_n3(fb):
        blocks = fb.reshape(NGRP, 4, 128)
        return blocks[:, :3, :].transpose(0, 2, 1).reshape(N_ATOMS, 3)

    return (e0, _to_n3(f0b), e1, _to_n3(f1b))
